# Initial kernel scaffold; baseline (speedup 1.0000x reference)
import jax
import jax.numpy as jnp
from jax.experimental import pallas as pl


def kernel(x, edge_index, line_graph_val):
    raise NotImplementedError("write your pallas kernel here")


# speedup vs baseline: 21.5961x; 21.5961x over previous
import functools

import jax
import jax.numpy as jnp
from jax import lax
from jax.experimental import pallas as pl
from jax.experimental.pallas import tpu as pltpu
from jax.experimental.pallas import tpu_sc as plsc

N_NODES = 10000
N_EDGES = 320000
D = 128

NC = 2
NS = 16
NW = NC * NS
K = 128
CH = 80
E_PAD = NW * CH * K
NPAD = 10240
DEGW = 16
RPW = NPAD // NS
BR = 256

_mesh = lambda: plsc.VectorSubcoreMesh(core_axis_name="c", subcore_axis_name="s")


def _deg_body(dst_hbm, zeros_hbm, ones_hbm, deg_out, dst_v, ones_v, deg_sh):
    c = lax.axis_index("c")
    s = lax.axis_index("s")
    w = s * NC + c
    pltpu.sync_copy(dst_hbm.at[w], dst_v)
    pltpu.sync_copy(ones_hbm, ones_v)
    pltpu.sync_copy(zeros_hbm.at[pl.ds(s * RPW, RPW)],
                    deg_sh.at[pl.ds(s * RPW, RPW)])
    plsc.subcore_barrier()

    def body(j, carry):
        pltpu.sync_copy(ones_v, deg_sh.at[dst_v.at[j]], add=True)
        return carry

    lax.fori_loop(0, CH, body, 0)
    plsc.subcore_barrier()
    pltpu.sync_copy(deg_sh.at[pl.ds(s * RPW, RPW)],
                    deg_out.at[c, pl.ds(s * RPW, RPW)])


@jax.jit
def _deg_kernel(dst3, zeros_deg, ones_k):
    return pl.kernel(
        _deg_body,
        out_type=jax.ShapeDtypeStruct((NC, NPAD, DEGW), jnp.float32),
        mesh=_mesh(),
        scratch_types=[
            pltpu.VMEM((CH, K), jnp.int32),
            pltpu.VMEM((K, DEGW), jnp.float32),
            pltpu.VMEM_SHARED((NPAD, DEGW), jnp.float32),
        ],
    )(dst3, zeros_deg, ones_k)


def _scale_body(x_ref, da_ref, db_ref, y_ref, dis_ref):
    d = da_ref[...] + db_ref[...]
    r = lax.rsqrt(d)
    y_ref[...] = x_ref[...] * r[:, 0:1]
    dis_ref[...] = jnp.where(d > 0.0, r, 0.0)


@jax.jit
def _scale_kernel(x_pad, deg_a, deg_b):
    return pl.pallas_call(
        _scale_body,
        grid=(NPAD // BR,),
        in_specs=[
            pl.BlockSpec((BR, D), lambda i: (i, 0)),
            pl.BlockSpec((BR, DEGW), lambda i: (i, 0)),
            pl.BlockSpec((BR, DEGW), lambda i: (i, 0)),
        ],
        out_specs=[
            pl.BlockSpec((BR, D), lambda i: (i, 0)),
            pl.BlockSpec((BR, DEGW), lambda i: (i, 0)),
        ],
        out_shape=[
            jax.ShapeDtypeStruct((NPAD, D), jnp.float32),
            jax.ShapeDtypeStruct((NPAD, DEGW), jnp.float32),
        ],
    )(x_pad, deg_a, deg_b)


def _agg_body(y_hbm, src_hbm, dst_hbm, zeros_hbm, acc_out,
              src_v, dst_v, rows_v, sem, acc_sh):
    c = lax.axis_index("c")
    s = lax.axis_index("s")
    w = s * NC + c
    pltpu.sync_copy(src_hbm.at[w], src_v)
    pltpu.sync_copy(dst_hbm.at[w], dst_v)
    pltpu.sync_copy(zeros_hbm.at[pl.ds(s * RPW, RPW)],
                    acc_sh.at[pl.ds(s * RPW, RPW)])
    plsc.subcore_barrier()

    def body(j, carry):
        pltpu.async_copy(y_hbm.at[src_v.at[j]], rows_v, sem).wait()
        pltpu.sync_copy(rows_v, acc_sh.at[dst_v.at[j]], add=True)
        return carry

    lax.fori_loop(0, CH, body, 0)
    plsc.subcore_barrier()
    pltpu.sync_copy(acc_sh.at[pl.ds(s * RPW, RPW)],
                    acc_out.at[c, pl.ds(s * RPW, RPW)])


@jax.jit
def _agg_kernel(y, src3, dst3, zeros_big):
    return pl.kernel(
        _agg_body,
        out_type=jax.ShapeDtypeStruct((NC, NPAD, D), jnp.float32),
        mesh=_mesh(),
        scratch_types=[
            pltpu.VMEM((CH, K), jnp.int32),
            pltpu.VMEM((CH, K), jnp.int32),
            pltpu.VMEM((K, D), jnp.float32),
            pltpu.SemaphoreType.DMA,
            pltpu.VMEM_SHARED((NPAD, D), jnp.float32),
        ],
    )(y, src3, dst3, zeros_big)


def _final_body(a_ref, b_ref, dis_ref, out_ref):
    acc = a_ref[...] + b_ref[...]
    out_ref[...] = jnp.maximum(acc * dis_ref[:, 0:1], 0.0)


@jax.jit
def _final_kernel(acc_a, acc_b, dis):
    return pl.pallas_call(
        _final_body,
        grid=(NPAD // BR,),
        in_specs=[
            pl.BlockSpec((BR, D), lambda i: (i, 0)),
            pl.BlockSpec((BR, D), lambda i: (i, 0)),
            pl.BlockSpec((BR, DEGW), lambda i: (i, 0)),
        ],
        out_specs=pl.BlockSpec((BR, D), lambda i: (i, 0)),
        out_shape=jax.ShapeDtypeStruct((NPAD, D), jnp.float32),
    )(acc_a, acc_b, dis)


@jax.jit
def kernel(x, edge_index, line_graph_val):
    src = edge_index[0].astype(jnp.int32)
    dst = edge_index[1].astype(jnp.int32)
    pad_n = E_PAD - N_EDGES
    pad_idx = (jnp.arange(pad_n, dtype=jnp.int32) % 16) + N_NODES
    src3 = jnp.concatenate([src, pad_idx]).reshape(NW, CH, K)
    dst3 = jnp.concatenate([dst, pad_idx]).reshape(NW, CH, K)
    x_pad = jnp.pad(x, ((0, NPAD - N_NODES), (0, 0)))

    zeros_deg = jnp.zeros((NPAD, DEGW), jnp.float32)
    ones_k = jnp.ones((K, DEGW), jnp.float32)
    zeros_big = jnp.zeros((NPAD, D), jnp.float32)

    deg_p = _deg_kernel(dst3, zeros_deg, ones_k)
    y, dis = _scale_kernel(x_pad, deg_p[0], deg_p[1])
    acc = _agg_kernel(y, src3, dst3, zeros_big)
    out = _final_kernel(acc[0], acc[1], dis)
    return out[:N_NODES]


# speedup vs baseline: 27.1692x; 1.2581x over previous
import functools

import jax
import jax.numpy as jnp
from jax import lax
from jax.experimental import pallas as pl
from jax.experimental.pallas import tpu as pltpu
from jax.experimental.pallas import tpu_sc as plsc

N_NODES = 10000
N_EDGES = 320000
D = 128

NC = 2
NS = 16
NW = NC * NS
K = 128
CH = 80
NP = 2
HCH = CH // NP
E_PAD = NW * CH * K
NPAD = 10240
DEGW = 16
RPW = NPAD // NS
BR = 200

_mesh = lambda: plsc.VectorSubcoreMesh(core_axis_name="c", subcore_axis_name="s")


def _deg_body(dst_hbm, zeros_hbm, ones_hbm, deg_out, dst_v, ones_v, deg_sem,
              deg_sh):
    c = lax.axis_index("c")
    s = lax.axis_index("s")
    w = s * NC + c
    pltpu.sync_copy(dst_hbm.at[w], dst_v)
    pltpu.sync_copy(ones_hbm, ones_v)
    pltpu.sync_copy(zeros_hbm.at[pl.ds(s * RPW, RPW)],
                    deg_sh.at[pl.ds(s * RPW, RPW)])
    plsc.subcore_barrier()

    def issue(j, carry):
        pltpu.async_copy(ones_v, deg_sh.at[dst_v.at[j]], deg_sem, add=True)
        return carry

    lax.fori_loop(0, CH, issue, 0)

    def drain(j, carry):
        pltpu.make_async_copy(ones_v, deg_sh.at[dst_v.at[0]], deg_sem).wait()
        return carry

    lax.fori_loop(0, CH, drain, 0)
    plsc.subcore_barrier()
    pltpu.sync_copy(deg_sh.at[pl.ds(s * RPW, RPW)],
                    deg_out.at[c, pl.ds(s * RPW, RPW)])


@jax.jit
def _deg_kernel(dst3, zeros_deg, ones_k):
    return pl.kernel(
        _deg_body,
        out_type=jax.ShapeDtypeStruct((NC, NPAD, DEGW), jnp.float32),
        mesh=_mesh(),
        scratch_types=[
            pltpu.VMEM((CH, K), jnp.int32),
            pltpu.VMEM((K, DEGW), jnp.float32),
            pltpu.SemaphoreType.DMA,
            pltpu.VMEM_SHARED((NPAD, DEGW), jnp.float32),
        ],
    )(dst3, zeros_deg, ones_k)


def _scale_body(x_ref, da_ref, db_ref, y_ref, dis_ref):
    d = da_ref[...] + db_ref[...]
    r = lax.rsqrt(d)
    y_ref[...] = x_ref[...] * r[:, 0:1]
    dis_ref[...] = jnp.where(d > 0.0, r, 0.0)


@jax.jit
def _scale_kernel(x, deg_a, deg_b):
    return pl.pallas_call(
        _scale_body,
        grid=(N_NODES // BR,),
        in_specs=[
            pl.BlockSpec((BR, D), lambda i: (i, 0)),
            pl.BlockSpec((BR, DEGW), lambda i: (i, 0)),
            pl.BlockSpec((BR, DEGW), lambda i: (i, 0)),
        ],
        out_specs=[
            pl.BlockSpec((BR, D), lambda i: (i, 0)),
            pl.BlockSpec((BR, DEGW), lambda i: (i, 0)),
        ],
        out_shape=[
            jax.ShapeDtypeStruct((NPAD, D), jnp.float32),
            jax.ShapeDtypeStruct((NPAD, DEGW), jnp.float32),
        ],
    )(x, deg_a, deg_b)


def _agg_body(y_hbm, src_hbm, dst_hbm, zeros_hbm, acc_out,
              src_v, dst_v, rows0, rows1, sem0, sem1, acc_sh):
    c = lax.axis_index("c")
    s = lax.axis_index("s")
    w = s * NC + c
    pltpu.sync_copy(zeros_hbm.at[pl.ds(s * RPW, RPW)],
                    acc_sh.at[pl.ds(s * RPW, RPW)])
    plsc.subcore_barrier()

    for p in range(NP):
        pltpu.sync_copy(src_hbm.at[w, pl.ds(p * HCH, HCH)], src_v)
        pltpu.sync_copy(dst_hbm.at[w, pl.ds(p * HCH, HCH)], dst_v)
        pltpu.async_copy(y_hbm.at[src_v.at[0]], rows0, sem0)

        def body(i, carry):
            j0 = 2 * i
            j1 = j0 + 1
            cp1 = pltpu.async_copy(y_hbm.at[src_v.at[j1]], rows1, sem1)
            pltpu.make_async_copy(y_hbm.at[src_v.at[j0]], rows0, sem0).wait()
            pltpu.sync_copy(rows0, acc_sh.at[dst_v.at[j0]], add=True)

            @pl.when(i < HCH // 2 - 1)
            def _():
                pltpu.async_copy(y_hbm.at[src_v.at[j0 + 2]], rows0, sem0)

            cp1.wait()
            pltpu.sync_copy(rows1, acc_sh.at[dst_v.at[j1]], add=True)
            return carry

        lax.fori_loop(0, HCH // 2, body, 0)
    plsc.subcore_barrier()
    pltpu.sync_copy(acc_sh.at[pl.ds(s * RPW, RPW)],
                    acc_out.at[c, pl.ds(s * RPW, RPW)])


@jax.jit
def _agg_kernel(y, src3, dst3, zeros_big):
    return pl.kernel(
        _agg_body,
        out_type=jax.ShapeDtypeStruct((NC, NPAD, D), jnp.float32),
        mesh=_mesh(),
        scratch_types=[
            pltpu.VMEM((HCH, K), jnp.int32),
            pltpu.VMEM((HCH, K), jnp.int32),
            pltpu.VMEM((K, D), jnp.float32),
            pltpu.VMEM((K, D), jnp.float32),
            pltpu.SemaphoreType.DMA,
            pltpu.SemaphoreType.DMA,
            pltpu.VMEM_SHARED((NPAD, D), jnp.float32),
        ],
    )(y, src3, dst3, zeros_big)


def _final_body(a_ref, b_ref, dis_ref, out_ref):
    acc = a_ref[...] + b_ref[...]
    out_ref[...] = jnp.maximum(acc * dis_ref[:, 0:1], 0.0)


@jax.jit
def _final_kernel(acc_a, acc_b, dis):
    return pl.pallas_call(
        _final_body,
        grid=(N_NODES // BR,),
        in_specs=[
            pl.BlockSpec((BR, D), lambda i: (i, 0)),
            pl.BlockSpec((BR, D), lambda i: (i, 0)),
            pl.BlockSpec((BR, DEGW), lambda i: (i, 0)),
        ],
        out_specs=pl.BlockSpec((BR, D), lambda i: (i, 0)),
        out_shape=jax.ShapeDtypeStruct((N_NODES, D), jnp.float32),
    )(acc_a, acc_b, dis)


@jax.jit
def kernel(x, edge_index, line_graph_val):
    src = edge_index[0].astype(jnp.int32)
    dst = edge_index[1].astype(jnp.int32)
    pad_n = E_PAD - N_EDGES
    pad_idx = (jnp.arange(pad_n, dtype=jnp.int32) % 16) + N_NODES
    src3 = jnp.concatenate([src, pad_idx]).reshape(NW, CH, K)
    dst3 = jnp.concatenate([dst, pad_idx]).reshape(NW, CH, K)

    zeros_deg = jnp.zeros((NPAD, DEGW), jnp.float32)
    ones_k = jnp.ones((K, DEGW), jnp.float32)
    zeros_big = jnp.zeros((NPAD, D), jnp.float32)

    deg_p = _deg_kernel(dst3, zeros_deg, ones_k)
    y, dis = _scale_kernel(x, deg_p[0], deg_p[1])
    acc = _agg_kernel(y, src3, dst3, zeros_big)
    return _final_kernel(acc[0], acc[1], dis)
